# Initial kernel scaffold; baseline (speedup 1.0000x reference)
import jax
import jax.numpy as jnp
from jax.experimental import pallas as pl


def kernel(x_q, edge_index_q, x_t, edge_index_t, W_q, b_q, W_t, b_t):
    raise NotImplementedError("write your pallas kernel here")


# speedup vs baseline: 2.6893x; 2.6893x over previous
import functools

import jax
import jax.numpy as jnp
from jax import lax
from jax.experimental import pallas as pl
from jax.experimental.pallas import tpu as pltpu
from jax.experimental.pallas import tpu_sc as plsc

_L = 16
_NT = 16
_NC = 2
_U = 128
_CH = 8


@functools.lru_cache(maxsize=None)
def _build_sc_agg(n, dh, k_units):
    rpt = -(-(n + 1) // _NT)
    rpt = -(-rpt // 8) * 8
    npad = rpt * _NT
    nfull = n // rpt
    tail = n - nfull * rpt

    mesh = plsc.VectorSubcoreMesh(core_axis_name="c", subcore_axis_name="s")

    @functools.partial(
        pl.kernel,
        out_type=[
            jax.ShapeDtypeStruct((_NC, n, dh), jnp.float32),
            jax.ShapeDtypeStruct((_NC, n, _L), jnp.float32),
        ],
        mesh=mesh,
        compiler_params=pltpu.CompilerParams(use_tc_tiling_on_sc=False),
        scratch_types=[
            pltpu.VMEM((_CH, _U), jnp.int32),
            pltpu.VMEM((_CH, _U), jnp.int32),
            pltpu.VMEM((_U, dh), jnp.float32),
            pltpu.VMEM((_U, _L), jnp.float32),
            pltpu.VMEM_SHARED((npad, dh), jnp.float32),
            pltpu.VMEM_SHARED((npad, _L), jnp.float32),
            pltpu.SemaphoreType.DMA,
        ],
    )
    def sc_agg(xs_hbm, src_hbm, dst_hbm, agg_hbm, deg_hbm,
               src_v, dst_v, rows_v, ones_v, agg_sh, deg_sh, sem):
        cid = lax.axis_index("c")
        sid = lax.axis_index("s")
        base = sid * rpt
        z16 = jnp.zeros((_L,), jnp.float32)
        o16 = jnp.ones((_L,), jnp.float32)

        def fill_row(i, carry):
            for kk in range(dh // _L):
                rows_v[i, pl.ds(kk * _L, _L)] = z16
            ones_v[i] = o16
            return carry
        lax.fori_loop(0, _U, fill_row, 0)

        for q in range(rpt // _U):
            pltpu.sync_copy(rows_v, agg_sh.at[pl.ds(base + q * _U, _U)])
        rtail = rpt - (rpt // _U) * _U
        if rtail:
            pltpu.sync_copy(rows_v.at[pl.ds(0, rtail)],
                            agg_sh.at[pl.ds(base + (rpt // _U) * _U, rtail)])
        zsrc = rows_v.at[pl.ds(0, _U), pl.ds(0, _L)]
        for q in range(rpt // _U):
            pltpu.sync_copy(zsrc, deg_sh.at[pl.ds(base + q * _U, _U)])
        if rtail:
            pltpu.sync_copy(rows_v.at[pl.ds(0, rtail), pl.ds(0, _L)],
                            deg_sh.at[pl.ds(base + (rpt // _U) * _U, rtail)])

        plsc.subcore_barrier()

        ubase = sid * k_units
        cvec = jnp.full((_L,), 0, jnp.int32) + cid

        def chunk(m, carry):
            pltpu.sync_copy(src_hbm.at[pl.ds(ubase + m * _CH, _CH)], src_v)
            pltpu.sync_copy(dst_hbm.at[pl.ds(ubase + m * _CH, _CH)], dst_v)

            def xform(i, c2):
                for kk in range(_U // _L):
                    v = src_v[i, pl.ds(kk * _L, _L)]
                    src_v[i, pl.ds(kk * _L, _L)] = v + v + cvec
                return c2
            lax.fori_loop(0, _CH, xform, 0)

            for j in range(_CH):
                gidx = src_v.at[j]
                didx = dst_v.at[j]
                pltpu.async_copy(xs_hbm.at[gidx], rows_v, sem).wait()
                pltpu.sync_copy(rows_v, agg_sh.at[didx], add=True)

                @pl.when(lax.rem(jnp.int32(j), 2) == cid)
                def _():
                    pltpu.sync_copy(ones_v, deg_sh.at[didx], add=True)
            return carry
        lax.fori_loop(0, k_units // _CH, chunk, 0)

        plsc.subcore_barrier()

        @pl.when(sid < nfull)
        def _():
            pltpu.sync_copy(agg_sh.at[pl.ds(base, rpt)],
                            agg_hbm.at[cid, pl.ds(base, rpt)])
            pltpu.sync_copy(deg_sh.at[pl.ds(base, rpt)],
                            deg_hbm.at[cid, pl.ds(base, rpt)])

        if tail:
            @pl.when(sid == nfull)
            def _():
                pltpu.sync_copy(agg_sh.at[pl.ds(base, tail)],
                                agg_hbm.at[cid, pl.ds(base, tail)])
                pltpu.sync_copy(deg_sh.at[pl.ds(base, tail)],
                                deg_hbm.at[cid, pl.ds(base, tail)])

    return sc_agg


def _sc_aggregate(x, edge_index):
    n, d = x.shape
    dh = d // 2
    e = edge_index.shape[1]
    k_units = -(-e // (_NT * _U))
    k_units = -(-k_units // _CH) * _CH
    ep = k_units * _NT * _U
    src = edge_index[0]
    dst = edge_index[1]
    src_p = jnp.concatenate([src, jnp.zeros((ep - e,), src.dtype)])
    dst_p = jnp.concatenate([dst, jnp.full((ep - e,), n, dst.dtype)])
    xs = x.reshape(2 * n, dh)
    agg, deg = _build_sc_agg(n, dh, k_units)(
        xs,
        src_p.astype(jnp.int32).reshape(-1, _U),
        dst_p.astype(jnp.int32).reshape(-1, _U),
    )
    return agg, deg


def _mlp_body(agg0_ref, agg1_ref, deg0_ref, deg1_ref, w_ref, b_ref, out_ref):
    dh = agg0_ref.shape[2]
    deg = (jnp.sum(deg0_ref[0], axis=1) + jnp.sum(deg1_ref[0], axis=1)) * (1.0 / _L)
    r = 1.0 / jnp.maximum(deg, 1.0)
    h0 = agg0_ref[0] * r[:, None]
    h1 = agg1_ref[0] * r[:, None]
    y = (jnp.dot(h0, w_ref[0:dh, :], preferred_element_type=jnp.float32)
         + jnp.dot(h1, w_ref[dh:, :], preferred_element_type=jnp.float32)
         + b_ref[...])
    out_ref[...] = jnp.maximum(y, 0.0)


@functools.lru_cache(maxsize=None)
def _build_mlp(n, d, rows):
    grid = (n // rows,)
    return pl.pallas_call(
        _mlp_body,
        grid=grid,
        in_specs=[
            pl.BlockSpec((1, rows, d // 2), lambda i: (0, i, 0)),
            pl.BlockSpec((1, rows, d // 2), lambda i: (1, i, 0)),
            pl.BlockSpec((1, rows, _L), lambda i: (0, i, 0)),
            pl.BlockSpec((1, rows, _L), lambda i: (1, i, 0)),
            pl.BlockSpec((d, d), lambda i: (0, 0)),
            pl.BlockSpec((1, d), lambda i: (0, 0)),
        ],
        out_specs=pl.BlockSpec((rows, d), lambda i: (i, 0)),
        out_shape=jax.ShapeDtypeStruct((n, d), jnp.float32),
    )


def _mlp(agg, deg, w, b):
    _, n, dh = agg.shape
    d = 2 * dh
    return _build_mlp(n, d, 1000)(agg, agg, deg, deg, w, b.reshape(1, d))


def kernel(x_q, edge_index_q, x_t, edge_index_t, W_q, b_q, W_t, b_t):
    agg_q, deg_q = _sc_aggregate(x_q, edge_index_q)
    agg_t, deg_t = _sc_aggregate(x_t, edge_index_t)
    out_q = _mlp(agg_q, deg_q, W_q, b_q)
    out_t = _mlp(agg_t, deg_t, W_t, b_t)
    return out_q, out_t


# speedup vs baseline: 3.2138x; 1.1950x over previous
import functools

import jax
import jax.numpy as jnp
from jax import lax
from jax.experimental import pallas as pl
from jax.experimental.pallas import tpu as pltpu
from jax.experimental.pallas import tpu_sc as plsc

_L = 16
_NT = 16
_NC = 2
_U = 128
_CH = 8


@functools.lru_cache(maxsize=None)
def _build_sc_agg(n, dh, k_units):
    rpt = -(-(n + 1) // _NT)
    rpt = -(-rpt // 8) * 8
    npad = rpt * _NT
    nfull = n // rpt
    tail = n - nfull * rpt

    mesh = plsc.VectorSubcoreMesh(core_axis_name="c", subcore_axis_name="s")

    @functools.partial(
        pl.kernel,
        out_type=[
            jax.ShapeDtypeStruct((_NC, n, dh), jnp.float32),
            jax.ShapeDtypeStruct((_NC, n, _L), jnp.float32),
        ],
        mesh=mesh,
        compiler_params=pltpu.CompilerParams(use_tc_tiling_on_sc=False),
        scratch_types=[
            pltpu.VMEM((_CH, _U), jnp.int32),
            pltpu.VMEM((_CH, _U), jnp.int32),
            pltpu.VMEM((_CH, _U), jnp.int32),
            pltpu.VMEM((_CH, _U), jnp.int32),
            pltpu.VMEM((_U, dh), jnp.float32),
            pltpu.VMEM((_U, dh), jnp.float32),
            pltpu.VMEM((_U, _L), jnp.float32),
            pltpu.VMEM_SHARED((npad, dh), jnp.float32),
            pltpu.VMEM_SHARED((npad, _L), jnp.float32),
            pltpu.SemaphoreType.DMA,
            pltpu.SemaphoreType.DMA,
        ],
    )
    def sc_agg(xs_hbm, src_hbm, dst_hbm, agg_hbm, deg_hbm,
               src_a, dst_a, src_b, dst_b, rows_a, rows_b, ones_v,
               agg_sh, deg_sh, sem_a, sem_b):
        rows_v = rows_a
        cid = lax.axis_index("c")
        sid = lax.axis_index("s")
        base = sid * rpt
        z16 = jnp.zeros((_L,), jnp.float32)
        o16 = jnp.ones((_L,), jnp.float32)

        def fill_row(i, carry):
            for kk in range(dh // _L):
                rows_v[i, pl.ds(kk * _L, _L)] = z16
            ones_v[i] = o16
            return carry
        lax.fori_loop(0, _U, fill_row, 0)

        for q in range(rpt // _U):
            pltpu.sync_copy(rows_v, agg_sh.at[pl.ds(base + q * _U, _U)])
        rtail = rpt - (rpt // _U) * _U
        if rtail:
            pltpu.sync_copy(rows_v.at[pl.ds(0, rtail)],
                            agg_sh.at[pl.ds(base + (rpt // _U) * _U, rtail)])
        zsrc = rows_v.at[pl.ds(0, _U), pl.ds(0, _L)]
        for q in range(rpt // _U):
            pltpu.sync_copy(zsrc, deg_sh.at[pl.ds(base + q * _U, _U)])
        if rtail:
            pltpu.sync_copy(rows_v.at[pl.ds(0, rtail), pl.ds(0, _L)],
                            deg_sh.at[pl.ds(base + (rpt // _U) * _U, rtail)])

        plsc.subcore_barrier()

        ubase = sid * k_units
        cvec = jnp.full((_L,), 0, jnp.int32) + cid
        npair = k_units // (2 * _CH)
        rows = (rows_a, rows_b)
        sems = (sem_a, sem_b)
        srcs = (src_a, src_b)
        dsts = (dst_a, dst_b)

        def load_idx(c, s_v, d_v):
            pltpu.sync_copy(src_hbm.at[pl.ds(ubase + c * _CH, _CH)], s_v)
            pltpu.sync_copy(dst_hbm.at[pl.ds(ubase + c * _CH, _CH)], d_v)

            def xf(i, c2):
                for kk in range(_U // _L):
                    v = s_v[i, pl.ds(kk * _L, _L)]
                    s_v[i, pl.ds(kk * _L, _L)] = v + v + cvec
                return c2
            lax.fori_loop(0, _CH, xf, 0)

        load_idx(0, src_a, dst_a)
        pltpu.async_copy(xs_hbm.at[src_a.at[0]], rows_a, sem_a)

        plsc.subcore_barrier()

        def pair(m, carry):
            load_idx(2 * m + 1, src_b, dst_b)
            for u in range(2 * _CH):
                half, j = divmod(u, _CH)
                cur, csem = rows[u % 2], sems[u % 2]
                nxt, nsem = rows[(u + 1) % 2], sems[(u + 1) % 2]
                if u < 2 * _CH - 1:
                    nhalf, nj = divmod(u + 1, _CH)
                    pltpu.async_copy(xs_hbm.at[srcs[nhalf].at[nj]], nxt, nsem)
                else:
                    @pl.when(m != npair - 1)
                    def _():
                        pltpu.async_copy(xs_hbm.at[src_a.at[0]], nxt, nsem)
                gidx = srcs[half].at[j]
                didx = dsts[half].at[j]
                pltpu.make_async_copy(xs_hbm.at[gidx], cur, csem).wait()
                pltpu.sync_copy(cur, agg_sh.at[didx], add=True)

                @pl.when((u % 2) == cid)
                def _():
                    pltpu.sync_copy(ones_v, deg_sh.at[didx], add=True)

                if u == _CH - 1:
                    @pl.when(m != npair - 1)
                    def _():
                        load_idx(2 * m + 2, src_a, dst_a)
            return carry
        lax.fori_loop(0, npair, pair, 0)

        plsc.subcore_barrier()

        @pl.when(sid < nfull)
        def _():
            pltpu.sync_copy(agg_sh.at[pl.ds(base, rpt)],
                            agg_hbm.at[cid, pl.ds(base, rpt)])
            pltpu.sync_copy(deg_sh.at[pl.ds(base, rpt)],
                            deg_hbm.at[cid, pl.ds(base, rpt)])

        if tail:
            @pl.when(sid == nfull)
            def _():
                pltpu.sync_copy(agg_sh.at[pl.ds(base, tail)],
                                agg_hbm.at[cid, pl.ds(base, tail)])
                pltpu.sync_copy(deg_sh.at[pl.ds(base, tail)],
                                deg_hbm.at[cid, pl.ds(base, tail)])

    return sc_agg


def _sc_aggregate(x, edge_index):
    n, d = x.shape
    dh = d // 2
    e = edge_index.shape[1]
    k_units = -(-e // (_NT * _U))
    k_units = -(-k_units // (2 * _CH)) * (2 * _CH)
    ep = k_units * _NT * _U
    src = edge_index[0]
    dst = edge_index[1]
    src_p = jnp.concatenate([src, jnp.zeros((ep - e,), src.dtype)])
    dst_p = jnp.concatenate([dst, jnp.full((ep - e,), n, dst.dtype)])
    xs = x.reshape(2 * n, dh)
    agg, deg = _build_sc_agg(n, dh, k_units)(
        xs,
        src_p.astype(jnp.int32).reshape(-1, _U),
        dst_p.astype(jnp.int32).reshape(-1, _U),
    )
    return agg, deg


def _mlp_body(agg0_ref, agg1_ref, deg0_ref, deg1_ref, w_ref, b_ref, out_ref):
    dh = agg0_ref.shape[2]
    deg = (jnp.sum(deg0_ref[0], axis=1) + jnp.sum(deg1_ref[0], axis=1)) * (1.0 / _L)
    r = 1.0 / jnp.maximum(deg, 1.0)
    h0 = agg0_ref[0] * r[:, None]
    h1 = agg1_ref[0] * r[:, None]
    y = (jnp.dot(h0, w_ref[0:dh, :], preferred_element_type=jnp.float32)
         + jnp.dot(h1, w_ref[dh:, :], preferred_element_type=jnp.float32)
         + b_ref[...])
    out_ref[...] = jnp.maximum(y, 0.0)


@functools.lru_cache(maxsize=None)
def _build_mlp(n, d, rows):
    grid = (n // rows,)
    return pl.pallas_call(
        _mlp_body,
        grid=grid,
        in_specs=[
            pl.BlockSpec((1, rows, d // 2), lambda i: (0, i, 0)),
            pl.BlockSpec((1, rows, d // 2), lambda i: (1, i, 0)),
            pl.BlockSpec((1, rows, _L), lambda i: (0, i, 0)),
            pl.BlockSpec((1, rows, _L), lambda i: (1, i, 0)),
            pl.BlockSpec((d, d), lambda i: (0, 0)),
            pl.BlockSpec((1, d), lambda i: (0, 0)),
        ],
        out_specs=pl.BlockSpec((rows, d), lambda i: (i, 0)),
        out_shape=jax.ShapeDtypeStruct((n, d), jnp.float32),
    )


def _mlp(agg, deg, w, b):
    _, n, dh = agg.shape
    d = 2 * dh
    return _build_mlp(n, d, 1000)(agg, agg, deg, deg, w, b.reshape(1, d))


def kernel(x_q, edge_index_q, x_t, edge_index_t, W_q, b_q, W_t, b_t):
    agg_q, deg_q = _sc_aggregate(x_q, edge_index_q)
    agg_t, deg_t = _sc_aggregate(x_t, edge_index_t)
    out_q = _mlp(agg_q, deg_q, W_q, b_q)
    out_t = _mlp(agg_t, deg_t, W_t, b_t)
    return out_q, out_t
